# baseline (speedup 1.0000x reference)
import functools

import jax
import jax.numpy as jnp
from jax import lax
from jax.experimental import pallas as pl
from jax.experimental.pallas import tpu as pltpu
from jax.experimental.pallas import tpu_sc as plsc

N = 10000
E = 160000
D = 256
HALF = 128

ROW_BLK = 1024
NPAD = 10240
N_BLKS = NPAD // ROW_BLK

CHUNK = 64
NBUF = 4
TILES = 16
EPAD = 163840
CH_SC = EPAD // TILES // CHUNK
CH_STG = CH_SC // 4
DCH = 128
CH_DEG = EPAD // (2 * TILES) // DCH
ROWS_T = NPAD // TILES


def _deg_body(col3, zeros_hbm, ones_hbm, deg_out, col_vm, ones_vm, acc_sh):
    c = lax.axis_index("c")
    s = lax.axis_index("s")
    pltpu.sync_copy(zeros_hbm.at[pl.ds(s * ROWS_T, ROWS_T)],
                    acc_sh.at[pl.ds(s * ROWS_T, ROWS_T)])
    pltpu.sync_copy(ones_hbm, ones_vm)
    pltpu.sync_copy(col3.at[c * TILES + s], col_vm)
    plsc.subcore_barrier()

    def body(j, carry):
        pltpu.sync_copy(ones_vm, acc_sh.at[col_vm.at[j]], add=True)
        return carry

    lax.fori_loop(0, CH_DEG, body, 0)
    plsc.subcore_barrier()
    pltpu.sync_copy(acc_sh.at[pl.ds(s * ROWS_T, ROWS_T)],
                    deg_out.at[c, pl.ds(s * ROWS_T, ROWS_T)])


@functools.cache
def _deg_call():
    mesh = plsc.VectorSubcoreMesh(core_axis_name="c", subcore_axis_name="s")
    return pl.kernel(
        _deg_body,
        out_type=jax.ShapeDtypeStruct((2, NPAD, HALF), jnp.float32),
        mesh=mesh,
        scratch_types=[
            pltpu.VMEM((CH_DEG, DCH), jnp.int32),
            pltpu.VMEM((DCH, HALF), jnp.float32),
            pltpu.VMEM_SHARED((NPAD, HALF), jnp.float32),
        ],
    )


def _scatter_body(g_flat, row3, col3, acc_out, row_vm, col_vm,
                  buf0, buf1, buf2, buf3, sem0, sem1, sem2, sem3, acc_sh):
    c = lax.axis_index("c")
    s = lax.axis_index("s")
    base = c * NPAD
    pltpu.sync_copy(g_flat.at[pl.ds(base + s * ROWS_T, ROWS_T)],
                    acc_sh.at[pl.ds(s * ROWS_T, ROWS_T)])
    plsc.subcore_barrier()

    ring = ((buf0, sem0), (buf1, sem1), (buf2, sem2), (buf3, sem3))

    def body(j, carry):
        base = NBUF * j
        for k, (buf, sem) in enumerate(ring):
            ck = base + k
            pltpu.make_async_copy(g_flat.at[row_vm.at[ck]], buf, sem).wait()
            pltpu.sync_copy(buf, acc_sh.at[col_vm.at[ck]], add=True)

            @pl.when(ck + NBUF < CH_STG)
            def _():
                pltpu.async_copy(g_flat.at[row_vm.at[ck + NBUF]], buf, sem)

        return carry

    for h in range(CH_SC // CH_STG):
        pltpu.sync_copy(row3.at[c, s, pl.ds(h * CH_STG, CH_STG)], row_vm)
        pltpu.sync_copy(col3.at[s, pl.ds(h * CH_STG, CH_STG)], col_vm)
        for k, (buf, sem) in enumerate(ring):
            pltpu.async_copy(g_flat.at[row_vm.at[k]], buf, sem)
        lax.fori_loop(0, CH_STG // NBUF, body, 0)
    plsc.subcore_barrier()
    pltpu.sync_copy(acc_sh.at[pl.ds(s * ROWS_T, ROWS_T)],
                    acc_out.at[pl.ds(base + s * ROWS_T, ROWS_T)])


@functools.cache
def _scatter_call():
    mesh = plsc.VectorSubcoreMesh(core_axis_name="c", subcore_axis_name="s")
    return pl.kernel(
        _scatter_body,
        out_type=jax.ShapeDtypeStruct((2 * NPAD, HALF), jnp.float32),
        mesh=mesh,
        scratch_types=[
            pltpu.VMEM((CH_STG, CHUNK), jnp.int32),
            pltpu.VMEM((CH_STG, CHUNK), jnp.int32),
            pltpu.VMEM((CHUNK, HALF), jnp.float32),
            pltpu.VMEM((CHUNK, HALF), jnp.float32),
            pltpu.VMEM((CHUNK, HALF), jnp.float32),
            pltpu.VMEM((CHUNK, HALF), jnp.float32),
            pltpu.SemaphoreType.DMA,
            pltpu.SemaphoreType.DMA,
            pltpu.SemaphoreType.DMA,
            pltpu.SemaphoreType.DMA,
            pltpu.VMEM_SHARED((NPAD, HALF), jnp.float32),
        ],
    )


def _dinv_from(degp_ref):
    deg = degp_ref[0, :, 0:1] + degp_ref[1, :, 0:1] + 1.0
    return lax.rsqrt(deg)


def _mm_first_body(x_ref, w_ref, degp_ref, g_ref):
    dinv = _dinv_from(degp_ref)
    h = jnp.dot(x_ref[...], w_ref[...], preferred_element_type=jnp.float32)
    g = h * dinv
    g_ref[0] = g[:, :HALF]
    g_ref[1] = g[:, HALF:]


def _mm_mid_body(acc_ref, degp_ref, b_ref, w_ref, g_ref):
    dinv = _dinv_from(degp_ref)
    t = jnp.concatenate([acc_ref[0], acc_ref[1]], axis=1) * dinv + b_ref[...]
    t = jnp.maximum(t, 0.0)
    h = jnp.dot(t, w_ref[...], preferred_element_type=jnp.float32)
    g = h * dinv
    g_ref[0] = g[:, :HALF]
    g_ref[1] = g[:, HALF:]


def _mm_final_body(acc_ref, degp_ref, b_ref, out_ref):
    dinv = _dinv_from(degp_ref)
    out_ref[...] = (jnp.concatenate([acc_ref[0], acc_ref[1]], axis=1) * dinv
                    + b_ref[...])


_degp_spec = pl.BlockSpec((2, ROW_BLK, HALF), lambda i: (0, i, 0))
_acc_spec = pl.BlockSpec((2, ROW_BLK, HALF), lambda i: (0, i, 0))
_g_spec = pl.BlockSpec((2, ROW_BLK, HALF), lambda i: (0, i, 0))
_w_spec = pl.BlockSpec((D, D), lambda i: (0, 0))
_b_spec = pl.BlockSpec((1, D), lambda i: (0, 0))

_mm_first = pl.pallas_call(
    _mm_first_body,
    grid=(N_BLKS,),
    in_specs=[pl.BlockSpec((ROW_BLK, D), lambda i: (i, 0)), _w_spec, _degp_spec],
    out_specs=_g_spec,
    out_shape=jax.ShapeDtypeStruct((2, NPAD, HALF), jnp.float32),
)

_mm_mid = pl.pallas_call(
    _mm_mid_body,
    grid=(N_BLKS,),
    in_specs=[_acc_spec, _degp_spec, _b_spec, _w_spec],
    out_specs=_g_spec,
    out_shape=jax.ShapeDtypeStruct((2, NPAD, HALF), jnp.float32),
)

_mm_final = pl.pallas_call(
    _mm_final_body,
    grid=(N_BLKS,),
    in_specs=[_acc_spec, _degp_spec, _b_spec],
    out_specs=pl.BlockSpec((ROW_BLK, D), lambda i: (i, 0)),
    out_shape=jax.ShapeDtypeStruct((NPAD, D), jnp.float32),
)


def kernel(x, edge_index, batch, W1, b1, W2, b2, W3, b3):
    del batch
    row = edge_index[0]
    col = edge_index[1]
    pad = EPAD - E
    rowp = jnp.concatenate([row, jnp.zeros((pad,), jnp.int32)])
    colp = jnp.concatenate([col, jnp.full((pad,), N, jnp.int32)])

    col_deg3 = colp.reshape(2 * TILES, CH_DEG, DCH)
    col3 = colp.reshape(TILES, CH_SC, CHUNK)
    row3 = (rowp.reshape(1, TILES, CH_SC, CHUNK)
            + jnp.array([0, NPAD], jnp.int32).reshape(2, 1, 1, 1))

    zeros_h = jnp.zeros((NPAD, HALF), jnp.float32)
    ones_h = jnp.ones((DCH, HALF), jnp.float32)
    degp = _deg_call()(col_deg3, zeros_h, ones_h)

    x_pad = jnp.concatenate([x, jnp.zeros((NPAD - N, D), x.dtype)], axis=0)
    b1r = b1.reshape(1, D)
    b2r = b2.reshape(1, D)
    b3r = b3.reshape(1, D)

    scatter = _scatter_call()
    g1 = _mm_first(x_pad, W1, degp)
    acc1 = scatter(g1.reshape(2 * NPAD, HALF), row3, col3)
    g2 = _mm_mid(acc1.reshape(2, NPAD, HALF), degp, b1r, W2)
    acc2 = scatter(g2.reshape(2 * NPAD, HALF), row3, col3)
    g3 = _mm_mid(acc2.reshape(2, NPAD, HALF), degp, b2r, W3)
    acc3 = scatter(g3.reshape(2 * NPAD, HALF), row3, col3)
    out = _mm_final(acc3.reshape(2, NPAD, HALF), degp, b3r)
    return out[:N]


# speedup vs baseline: 1.0314x; 1.0314x over previous
import functools

import jax
import jax.numpy as jnp
from jax import lax
from jax.experimental import pallas as pl
from jax.experimental.pallas import tpu as pltpu
from jax.experimental.pallas import tpu_sc as plsc

N = 10000
E = 160000
D = 256
HALF = 128

ROW_BLK = 1024
NPAD = 10240
N_BLKS = NPAD // ROW_BLK

CHUNK = 128
NBUF = 2
TILES = 16
EPAD = 163840
CH_SC = EPAD // TILES // CHUNK
CH_STG = CH_SC // 2
DCH = 128
DW = 128
CH_DEG = EPAD // (2 * TILES) // DCH
ROWS_T = NPAD // TILES


def _deg_body(col3, zeros_hbm, ones_hbm, deg_out, col_vm, ones_vm, acc_sh):
    c = lax.axis_index("c")
    s = lax.axis_index("s")
    pltpu.sync_copy(zeros_hbm.at[pl.ds(s * ROWS_T, ROWS_T)],
                    acc_sh.at[pl.ds(s * ROWS_T, ROWS_T)])
    pltpu.sync_copy(ones_hbm, ones_vm)
    pltpu.sync_copy(col3.at[c * TILES + s], col_vm)
    plsc.subcore_barrier()

    def body(j, carry):
        pltpu.sync_copy(ones_vm, acc_sh.at[col_vm.at[j]], add=True)
        return carry

    lax.fori_loop(0, CH_DEG, body, 0)
    plsc.subcore_barrier()
    pltpu.sync_copy(acc_sh.at[pl.ds(s * ROWS_T, ROWS_T)],
                    deg_out.at[c, pl.ds(s * ROWS_T, ROWS_T)])


@functools.cache
def _deg_call():
    mesh = plsc.VectorSubcoreMesh(core_axis_name="c", subcore_axis_name="s")
    return pl.kernel(
        _deg_body,
        out_type=jax.ShapeDtypeStruct((2, NPAD, DW), jnp.float32),
        mesh=mesh,
        scratch_types=[
            pltpu.VMEM((CH_DEG, DCH), jnp.int32),
            pltpu.VMEM((DCH, DW), jnp.float32),
            pltpu.VMEM_SHARED((NPAD, DW), jnp.float32),
        ],
    )


def _scatter_body(g_flat, row3, col3, acc_out, row_vm, col_vm,
                  buf0, buf1, sem0, sem1, acc_sh):
    c = lax.axis_index("c")
    s = lax.axis_index("s")
    base = c * NPAD
    pltpu.sync_copy(g_flat.at[pl.ds(base + s * ROWS_T, ROWS_T)],
                    acc_sh.at[pl.ds(s * ROWS_T, ROWS_T)])
    plsc.subcore_barrier()

    ring = ((buf0, sem0), (buf1, sem1))

    def body(j, carry):
        base = NBUF * j
        for k, (buf, sem) in enumerate(ring):
            ck = base + k
            pltpu.make_async_copy(g_flat.at[row_vm.at[ck]], buf, sem).wait()
            pltpu.sync_copy(buf, acc_sh.at[col_vm.at[ck]], add=True)

            @pl.when(ck + NBUF < CH_STG)
            def _():
                pltpu.async_copy(g_flat.at[row_vm.at[ck + NBUF]], buf, sem)

        return carry

    for h in range(CH_SC // CH_STG):
        pltpu.sync_copy(row3.at[c, s, pl.ds(h * CH_STG, CH_STG)], row_vm)
        pltpu.sync_copy(col3.at[s, pl.ds(h * CH_STG, CH_STG)], col_vm)
        for k, (buf, sem) in enumerate(ring):
            pltpu.async_copy(g_flat.at[row_vm.at[k]], buf, sem)
        lax.fori_loop(0, CH_STG // NBUF, body, 0)
    plsc.subcore_barrier()
    pltpu.sync_copy(acc_sh.at[pl.ds(s * ROWS_T, ROWS_T)],
                    acc_out.at[pl.ds(base + s * ROWS_T, ROWS_T)])


@functools.cache
def _scatter_call():
    mesh = plsc.VectorSubcoreMesh(core_axis_name="c", subcore_axis_name="s")
    return pl.kernel(
        _scatter_body,
        out_type=jax.ShapeDtypeStruct((2 * NPAD, HALF), jnp.float32),
        mesh=mesh,
        scratch_types=[
            pltpu.VMEM((CH_STG, CHUNK), jnp.int32),
            pltpu.VMEM((CH_STG, CHUNK), jnp.int32),
            pltpu.VMEM((CHUNK, HALF), jnp.float32),
            pltpu.VMEM((CHUNK, HALF), jnp.float32),
            pltpu.SemaphoreType.DMA,
            pltpu.SemaphoreType.DMA,
            pltpu.VMEM_SHARED((NPAD, HALF), jnp.float32),
        ],
    )


def _dinv_from(degp_ref):
    deg = degp_ref[0, :, 0:1] + degp_ref[1, :, 0:1] + 1.0
    return lax.rsqrt(deg)


def _mm_first_body(x_ref, w_ref, degp_ref, g_ref):
    dinv = _dinv_from(degp_ref)
    h = jnp.dot(x_ref[...], w_ref[...], preferred_element_type=jnp.float32)
    g = h * dinv
    g_ref[0] = g[:, :HALF]
    g_ref[1] = g[:, HALF:]


def _mm_mid_body(acc_ref, degp_ref, b_ref, w_ref, g_ref):
    dinv = _dinv_from(degp_ref)
    t = jnp.concatenate([acc_ref[0], acc_ref[1]], axis=1) * dinv + b_ref[...]
    t = jnp.maximum(t, 0.0)
    h = jnp.dot(t, w_ref[...], preferred_element_type=jnp.float32)
    g = h * dinv
    g_ref[0] = g[:, :HALF]
    g_ref[1] = g[:, HALF:]


def _mm_final_body(acc_ref, degp_ref, b_ref, out_ref):
    dinv = _dinv_from(degp_ref)
    out_ref[...] = (jnp.concatenate([acc_ref[0], acc_ref[1]], axis=1) * dinv
                    + b_ref[...])


_degp_spec = pl.BlockSpec((2, ROW_BLK, DW), lambda i: (0, i, 0))
_acc_spec = pl.BlockSpec((2, ROW_BLK, HALF), lambda i: (0, i, 0))
_g_spec = pl.BlockSpec((2, ROW_BLK, HALF), lambda i: (0, i, 0))
_w_spec = pl.BlockSpec((D, D), lambda i: (0, 0))
_b_spec = pl.BlockSpec((1, D), lambda i: (0, 0))

_mm_first = pl.pallas_call(
    _mm_first_body,
    grid=(N_BLKS,),
    in_specs=[pl.BlockSpec((ROW_BLK, D), lambda i: (i, 0)), _w_spec, _degp_spec],
    out_specs=_g_spec,
    out_shape=jax.ShapeDtypeStruct((2, NPAD, HALF), jnp.float32),
)

_mm_mid = pl.pallas_call(
    _mm_mid_body,
    grid=(N_BLKS,),
    in_specs=[_acc_spec, _degp_spec, _b_spec, _w_spec],
    out_specs=_g_spec,
    out_shape=jax.ShapeDtypeStruct((2, NPAD, HALF), jnp.float32),
)

_mm_final = pl.pallas_call(
    _mm_final_body,
    grid=(N_BLKS,),
    in_specs=[_acc_spec, _degp_spec, _b_spec],
    out_specs=pl.BlockSpec((ROW_BLK, D), lambda i: (i, 0)),
    out_shape=jax.ShapeDtypeStruct((NPAD, D), jnp.float32),
)


def kernel(x, edge_index, batch, W1, b1, W2, b2, W3, b3):
    del batch
    row = edge_index[0]
    col = edge_index[1]
    pad = EPAD - E
    rowp = jnp.concatenate([row, jnp.zeros((pad,), jnp.int32)])
    colp = jnp.concatenate([col, jnp.full((pad,), N, jnp.int32)])

    col_deg3 = colp.reshape(2 * TILES, CH_DEG, DCH)
    col3 = colp.reshape(TILES, CH_SC, CHUNK)
    row3 = (rowp.reshape(1, TILES, CH_SC, CHUNK)
            + jnp.array([0, NPAD], jnp.int32).reshape(2, 1, 1, 1))

    zeros_h = jnp.zeros((NPAD, DW), jnp.float32)
    ones_h = jnp.ones((DCH, DW), jnp.float32)
    degp = _deg_call()(col_deg3, zeros_h, ones_h)

    x_pad = jnp.concatenate([x, jnp.zeros((NPAD - N, D), x.dtype)], axis=0)
    b1r = b1.reshape(1, D)
    b2r = b2.reshape(1, D)
    b3r = b3.reshape(1, D)

    scatter = _scatter_call()
    g1 = _mm_first(x_pad, W1, degp)
    acc1 = scatter(g1.reshape(2 * NPAD, HALF), row3, col3)
    g2 = _mm_mid(acc1.reshape(2, NPAD, HALF), degp, b1r, W2)
    acc2 = scatter(g2.reshape(2 * NPAD, HALF), row3, col3)
    g3 = _mm_mid(acc2.reshape(2, NPAD, HALF), degp, b2r, W3)
    acc3 = scatter(g3.reshape(2 * NPAD, HALF), row3, col3)
    out = _mm_final(acc3.reshape(2, NPAD, HALF), degp, b3r)
    return out[:N]
